# baseline (speedup 1.0000x reference)
import jax
import jax.numpy as jnp
from jax.experimental import pallas as pl
from jax.experimental.pallas import tpu as pltpu

_D_MODEL = 2048
_N_EXP = 64
_TOP_K = 2
_ID_BIAS = -2.0
_TILE_T = 512
_D_SPLIT = 4
_D_CHUNK = _D_MODEL // _D_SPLIT


def _router_block(ha_ref, hb_ref, hc_ref, hd_ref, wt_ref, valid_ref, mask_ref, probs_ref):
    wt = wt_ref[...]
    logits = jnp.dot(ha_ref[...], wt[0 * _D_CHUNK:1 * _D_CHUNK, :],
                     preferred_element_type=jnp.float32)
    logits = logits + jnp.dot(hb_ref[...], wt[1 * _D_CHUNK:2 * _D_CHUNK, :],
                              preferred_element_type=jnp.float32)
    logits = logits + jnp.dot(hc_ref[...], wt[2 * _D_CHUNK:3 * _D_CHUNK, :],
                              preferred_element_type=jnp.float32)
    logits = logits + jnp.dot(hd_ref[...], wt[3 * _D_CHUNK:4 * _D_CHUNK, :],
                              preferred_element_type=jnp.float32)

    tile_t, n_exp = logits.shape
    col = jax.lax.broadcasted_iota(jnp.int32, (tile_t, n_exp), 1)
    id_col_f = jnp.where(col == (n_exp - 1), 1.0, 0.0)
    logits = logits + id_col_f * _ID_BIAS

    pad_f = 1.0 - valid_ref[...]
    pad_non_id = pad_f * (1.0 - id_col_f)
    logits = jnp.where(pad_non_id > 0.0, jnp.full_like(logits, -1e30), logits)
    logits = logits + (pad_f * id_col_f) * 1e30

    m1 = jnp.max(logits, axis=1, keepdims=True)
    e = jnp.exp(logits - m1)
    probs_ref[...] = e / jnp.sum(e, axis=1, keepdims=True)

    idx1 = jnp.argmax(logits, axis=1)[:, None]
    sans_top1 = jnp.where(col == idx1, jnp.full_like(logits, -jnp.inf), logits)
    idx2 = jnp.argmax(sans_top1, axis=1)[:, None]
    top2_f = jnp.where(col == idx1, 1.0, 0.0) + jnp.where(col == idx2, 1.0, 0.0)
    mask_ref[...] = pad_f * id_col_f + (1.0 - pad_f) * top2_f


def kernel(h, is_valid, W):
    t_tokens, d_model = h.shape
    n_exp = W.shape[0]
    wt = W.T
    valid = is_valid.astype(jnp.float32)[:, None]
    grid = (t_tokens // _TILE_T,)

    mask_f32, probs = pl.pallas_call(
        _router_block,
        grid=grid,
        in_specs=[
            pl.BlockSpec((_TILE_T, _D_CHUNK), lambda i: (i, 0)),
            pl.BlockSpec((_TILE_T, _D_CHUNK), lambda i: (i, 1)),
            pl.BlockSpec((_TILE_T, _D_CHUNK), lambda i: (i, 2)),
            pl.BlockSpec((_TILE_T, _D_CHUNK), lambda i: (i, 3)),
            pl.BlockSpec((d_model, n_exp), lambda i: (0, 0)),
            pl.BlockSpec((_TILE_T, 1), lambda i: (i, 0)),
        ],
        out_specs=[
            pl.BlockSpec((_TILE_T, n_exp), lambda i: (i, 0)),
            pl.BlockSpec((_TILE_T, n_exp), lambda i: (i, 0)),
        ],
        out_shape=[
            jax.ShapeDtypeStruct((t_tokens, n_exp), jnp.float32),
            jax.ShapeDtypeStruct((t_tokens, n_exp), jnp.float32),
        ],
        compiler_params=pltpu.CompilerParams(
            dimension_semantics=("parallel",),
        ),
    )(h, h, h, h, wt, valid)

    return (mask_f32.astype(bool), probs)


# speedup vs baseline: 1.1143x; 1.1143x over previous
import jax
import jax.numpy as jnp
from jax.experimental import pallas as pl
from jax.experimental.pallas import tpu as pltpu

_D_MODEL = 2048
_N_EXP = 64
_TOP_K = 2
_ID_BIAS = -2.0
_TILE_T = 2048
_D_SPLIT = 4
_D_CHUNK = _D_MODEL // _D_SPLIT


def _router_block(ha_ref, hb_ref, hc_ref, hd_ref, wt_ref, valid_ref, mask_ref, probs_ref):
    wt = wt_ref[...]
    logits = jnp.dot(ha_ref[...], wt[0 * _D_CHUNK:1 * _D_CHUNK, :],
                     preferred_element_type=jnp.float32)
    logits = logits + jnp.dot(hb_ref[...], wt[1 * _D_CHUNK:2 * _D_CHUNK, :],
                              preferred_element_type=jnp.float32)
    logits = logits + jnp.dot(hc_ref[...], wt[2 * _D_CHUNK:3 * _D_CHUNK, :],
                              preferred_element_type=jnp.float32)
    logits = logits + jnp.dot(hd_ref[...], wt[3 * _D_CHUNK:4 * _D_CHUNK, :],
                              preferred_element_type=jnp.float32)

    tile_t, n_exp = logits.shape
    col = jax.lax.broadcasted_iota(jnp.int32, (tile_t, n_exp), 1)
    id_col_f = jnp.where(col == (n_exp - 1), 1.0, 0.0)
    logits = logits + id_col_f * _ID_BIAS

    pad_f = 1.0 - valid_ref[...]
    pad_non_id = pad_f * (1.0 - id_col_f)
    logits = jnp.where(pad_non_id > 0.0, jnp.full_like(logits, -1e30), logits)
    logits = logits + (pad_f * id_col_f) * 1e30

    m1 = jnp.max(logits, axis=1, keepdims=True)
    e = jnp.exp(logits - m1)
    probs_ref[...] = e / jnp.sum(e, axis=1, keepdims=True)

    idx1 = jnp.argmax(logits, axis=1)[:, None]
    sans_top1 = jnp.where(col == idx1, jnp.full_like(logits, -jnp.inf), logits)
    idx2 = jnp.argmax(sans_top1, axis=1)[:, None]
    top2_f = jnp.where(col == idx1, 1.0, 0.0) + jnp.where(col == idx2, 1.0, 0.0)
    mask_ref[...] = pad_f * id_col_f + (1.0 - pad_f) * top2_f


def kernel(h, is_valid, W):
    t_tokens, d_model = h.shape
    n_exp = W.shape[0]
    wt = W.T
    valid = is_valid.astype(jnp.float32)[:, None]
    grid = (t_tokens // _TILE_T,)

    mask_f32, probs = pl.pallas_call(
        _router_block,
        grid=grid,
        in_specs=[
            pl.BlockSpec((_TILE_T, _D_CHUNK), lambda i: (i, 0)),
            pl.BlockSpec((_TILE_T, _D_CHUNK), lambda i: (i, 1)),
            pl.BlockSpec((_TILE_T, _D_CHUNK), lambda i: (i, 2)),
            pl.BlockSpec((_TILE_T, _D_CHUNK), lambda i: (i, 3)),
            pl.BlockSpec((d_model, n_exp), lambda i: (0, 0)),
            pl.BlockSpec((_TILE_T, 1), lambda i: (i, 0)),
        ],
        out_specs=[
            pl.BlockSpec((_TILE_T, n_exp), lambda i: (i, 0)),
            pl.BlockSpec((_TILE_T, n_exp), lambda i: (i, 0)),
        ],
        out_shape=[
            jax.ShapeDtypeStruct((t_tokens, n_exp), jnp.float32),
            jax.ShapeDtypeStruct((t_tokens, n_exp), jnp.float32),
        ],
        compiler_params=pltpu.CompilerParams(
            dimension_semantics=("parallel",),
        ),
    )(h, h, h, h, wt, valid)

    return (mask_f32.astype(bool), probs)


# speedup vs baseline: 1.1600x; 1.0410x over previous
import jax
import jax.numpy as jnp
from jax.experimental import pallas as pl
from jax.experimental.pallas import tpu as pltpu

_D_MODEL = 2048
_N_EXP = 64
_TOP_K = 2
_ID_BIAS = -2.0
_TILE_T = 1024
_D_SPLIT = 4
_D_CHUNK = _D_MODEL // _D_SPLIT


def _router_block(ha_ref, hb_ref, hc_ref, hd_ref, wt_ref, valid_ref, mask_ref, probs_ref):
    wt = wt_ref[...]
    logits = jnp.dot(ha_ref[...], wt[0 * _D_CHUNK:1 * _D_CHUNK, :],
                     preferred_element_type=jnp.float32)
    logits = logits + jnp.dot(hb_ref[...], wt[1 * _D_CHUNK:2 * _D_CHUNK, :],
                              preferred_element_type=jnp.float32)
    logits = logits + jnp.dot(hc_ref[...], wt[2 * _D_CHUNK:3 * _D_CHUNK, :],
                              preferred_element_type=jnp.float32)
    logits = logits + jnp.dot(hd_ref[...], wt[3 * _D_CHUNK:4 * _D_CHUNK, :],
                              preferred_element_type=jnp.float32)

    tile_t, n_exp = logits.shape
    col = jax.lax.broadcasted_iota(jnp.int32, (tile_t, n_exp), 1)
    id_col_f = jnp.where(col == (n_exp - 1), 1.0, 0.0)
    logits = logits + id_col_f * _ID_BIAS

    pad_f = 1.0 - valid_ref[...]
    pad_non_id = pad_f * (1.0 - id_col_f)
    logits = jnp.where(pad_non_id > 0.0, jnp.full_like(logits, -1e30), logits)
    logits = logits + (pad_f * id_col_f) * 1e30

    m1 = jnp.max(logits, axis=1, keepdims=True)
    e = jnp.exp(logits - m1)
    probs_ref[...] = e / jnp.sum(e, axis=1, keepdims=True)

    idx1 = jnp.argmax(logits, axis=1)[:, None]
    sans_top1 = jnp.where(col == idx1, jnp.full_like(logits, -jnp.inf), logits)
    idx2 = jnp.argmax(sans_top1, axis=1)[:, None]
    top2_f = jnp.where(col == idx1, 1.0, 0.0) + jnp.where(col == idx2, 1.0, 0.0)
    mask_f = pad_f * id_col_f + (1.0 - pad_f) * top2_f
    mask_ref[...] = mask_f.astype(jnp.int8)


def kernel(h, is_valid, W):
    t_tokens, d_model = h.shape
    n_exp = W.shape[0]
    wt = W.T
    valid = is_valid.astype(jnp.float32)[:, None]
    grid = (t_tokens // _TILE_T,)

    mask_i8, probs = pl.pallas_call(
        _router_block,
        grid=grid,
        in_specs=[
            pl.BlockSpec((_TILE_T, _D_CHUNK), lambda i: (i, 0)),
            pl.BlockSpec((_TILE_T, _D_CHUNK), lambda i: (i, 1)),
            pl.BlockSpec((_TILE_T, _D_CHUNK), lambda i: (i, 2)),
            pl.BlockSpec((_TILE_T, _D_CHUNK), lambda i: (i, 3)),
            pl.BlockSpec((d_model, n_exp), lambda i: (0, 0)),
            pl.BlockSpec((_TILE_T, 1), lambda i: (i, 0)),
        ],
        out_specs=[
            pl.BlockSpec((_TILE_T, n_exp), lambda i: (i, 0)),
            pl.BlockSpec((_TILE_T, n_exp), lambda i: (i, 0)),
        ],
        out_shape=[
            jax.ShapeDtypeStruct((t_tokens, n_exp), jnp.int8),
            jax.ShapeDtypeStruct((t_tokens, n_exp), jnp.float32),
        ],
        compiler_params=pltpu.CompilerParams(
            dimension_semantics=("parallel",),
        ),
    )(h, h, h, h, wt, valid)

    return (mask_i8.astype(bool), probs)
